# baseline (speedup 1.0000x reference)
import functools

import jax
import jax.numpy as jnp
from jax import lax
from jax.experimental import pallas as pl
from jax.experimental.pallas import tpu as pltpu
from jax.experimental.pallas import tpu_sc as plsc

NORM = 1.0
MEAN = 0.0
VOCAB_N = 1000000
EMBED_D = 64
NWORK = 32
PAIR_ROWS = VOCAB_N // 2
FULL_VBLKS = VOCAB_N // 128
MAIN_VBLKS = (FULL_VBLKS // NWORK) * NWORK
CHUNK = 128


def _iota16():
    return jnp.arange(16, dtype=jnp.int32)


def _make_transpose():
    mesh = plsc.VectorSubcoreMesh(core_axis_name="c", subcore_axis_name="s")

    NB = 4

    @functools.partial(
        pl.kernel,
        out_type=jax.ShapeDtypeStruct((PAIR_ROWS, 128), jnp.float32),
        mesh=mesh,
        scratch_types=[
            [pltpu.VMEM((64, 128), jnp.float32) for _ in range(NB)],
            [pltpu.VMEM((64, 128), jnp.float32) for _ in range(2)],
            [pltpu.SemaphoreType.DMA for _ in range(NB)],
            [pltpu.SemaphoreType.DMA for _ in range(2)],
        ],
        compiler_params=pltpu.CompilerParams(use_tc_tiling_on_sc=True, needs_layout_passes=False),
    )
    def k1(tbl_t, tail_t, w_out, bb, wb, bsem, wsem):
        wid = lax.axis_index("s") * 2 + lax.axis_index("c")
        rowv = tuple(_iota16() + (16 * m) for m in range(4))
        n_main = MAIN_VBLKS // NWORK

        def in_slice(t):
            off = pl.multiple_of((t * NWORK + wid) * 128, 128)
            return tbl_t.at[:, pl.ds(off, 128)]

        def out_slice(t):
            off = pl.multiple_of((t * NWORK + wid) * 64, 64)
            return w_out.at[pl.ds(off, 64), :]

        def start_in(t, s):
            pltpu.async_copy(in_slice(t), bb[s], bsem[s])
            return None

        prow = tuple((_iota16() + 16 * q) >> 1 for q in range(8))
        pcolb = tuple(((_iota16() + 16 * q) & 1) << 6 for q in range(8))

        def transpose_block(b, wb_):
            @plsc.parallel_loop(0, 64, unroll=8)
            def _(d):
                for q in range(8):
                    v = b[d, pl.ds(16 * q, 16)]
                    plsc.store_scatter(wb_, [prow[q], pcolb[q] + d], v)

        for t0 in range(NB):
            start_in(t0, t0)

        def body(tt, _):
            base = tt * NB
            for b in range(NB):
                t = base + b
                sw = b % 2
                pltpu.make_async_copy(in_slice(t), bb[b], bsem[b]).wait()
                if b >= 2:
                    pltpu.make_async_copy(wb[sw], out_slice(t - 2), wsem[sw]).wait()
                else:
                    lax.cond(
                        tt > 0,
                        lambda: pltpu.make_async_copy(wb[sw], out_slice(t - 2), wsem[sw]).wait(),
                        lambda: None,
                    )
                transpose_block(bb[b], wb[sw])
                lax.cond(t + NB < n_main, lambda: start_in(t + NB, b), lambda: None)
                pltpu.async_copy(wb[sw], out_slice(t), wsem[sw])
            return 0

        lax.fori_loop(0, n_main // NB, body, 0)
        pltpu.make_async_copy(wb[0], out_slice(n_main - 2), wsem[0]).wait()
        pltpu.make_async_copy(wb[1], out_slice(n_main - 1), wsem[1]).wait()

        @pl.when(wid < FULL_VBLKS - MAIN_VBLKS)
        def _():
            vblk = MAIN_VBLKS + wid
            pltpu.sync_copy(tbl_t.at[:, pl.ds(vblk * 128, 128)], bb[0])
            transpose_block(bb[0], wb[0])
            pltpu.sync_copy(wb[0], w_out.at[pl.ds(vblk * 64, 64), :])

        @pl.when(wid == 4)
        def _():
            pltpu.sync_copy(tail_t, bb[1])
            transpose_block(bb[1], wb[1])
            pltpu.sync_copy(wb[1].at[pl.ds(32, 32), :], w_out.at[pl.ds(FULL_VBLKS * 64, 32), :])

    return k1


def _make_gather(total: int):
    n_chunks = total // CHUNK
    per_worker = n_chunks // NWORK
    mesh = plsc.VectorSubcoreMesh(core_axis_name="c", subcore_axis_name="s")

    NG = 4
    NI = 8

    @functools.partial(
        pl.kernel,
        out_type=jax.ShapeDtypeStruct((26, EMBED_D, 16384), jnp.float32),
        mesh=mesh,
        scratch_types=[
            [pltpu.VMEM((CHUNK,), jnp.int32) for _ in range(NI)],
            [pltpu.VMEM((CHUNK,), jnp.int32) for _ in range(NG)],
            [pltpu.VMEM((CHUNK, 128), jnp.float32) for _ in range(NG)],
            [pltpu.VMEM((EMBED_D, 128), jnp.float32) for _ in range(2)],
            [pltpu.SemaphoreType.DMA for _ in range(NI)],
            [pltpu.SemaphoreType.DMA for _ in range(NG)],
            [pltpu.SemaphoreType.DMA for _ in range(2)],
        ],
        compiler_params=pltpu.CompilerParams(use_tc_tiling_on_sc=True, needs_layout_passes=False),
    )
    def k2(w_in, idx_in, p_out, ix, pr, g, pbuf, isem, gsem, psem):
        wid = lax.axis_index("s") * 2 + lax.axis_index("c")
        rowv = tuple(_iota16() + (16 * m) for m in range(4))

        def chunk_id(t):
            return t * NWORK + wid

        def idx_slice(t):
            off = pl.multiple_of(chunk_id(t) * CHUNK, CHUNK)
            return idx_in.at[pl.ds(off, CHUNK)]

        def out_slice(t):
            c = chunk_id(t)
            j = c // 128
            off = pl.multiple_of(lax.rem(c, 128) * 128, 128)
            return p_out.at[j, :, pl.ds(off, 128)]

        def start_idx(t, si):
            pltpu.async_copy(idx_slice(t), ix[si], isem[si])
            return None

        def start_gather(t, si, sg):
            pltpu.make_async_copy(idx_slice(t), ix[si], isem[si]).wait()
            for m in range(8):
                x = ix[si][pl.ds(16 * m, 16)]
                pr[sg][pl.ds(16 * m, 16)] = jnp.right_shift(x, 1)
            pltpu.async_copy(w_in.at[pr[sg]], g[sg], gsem[sg])
            return None

        def compute(si, sg, sp):
            colbase = []
            for m in range(8):
                x = ix[si][pl.ds(16 * m, 16)]
                colbase.append(jnp.left_shift(jnp.bitwise_and(x, 1), 6))

            @plsc.parallel_loop(0, EMBED_D, unroll=8)
            def _(d):
                for m in range(8):
                    rv = rowv[m % 4] + (64 if m >= 4 else 0)
                    v = plsc.load_gather(g[sg], [rv, colbase[m] + d])
                    pbuf[sp][d, pl.ds(16 * m, 16)] = v

        for t0 in range(NI):
            start_idx(t0, t0)
        for t0 in range(NG):
            start_gather(t0, t0, t0)

        n_outer = per_worker // NI

        def body(tt, _):
            base = tt * NI
            for b in range(NI):
                t = base + b
                si, sg, sp = b, b % NG, b % 2
                pltpu.make_async_copy(w_in.at[pr[sg]], g[sg], gsem[sg]).wait()
                if b >= 2:
                    pltpu.make_async_copy(pbuf[sp], out_slice(t - 2), psem[sp]).wait()
                else:
                    lax.cond(
                        tt > 0,
                        lambda: pltpu.make_async_copy(pbuf[sp], out_slice(t - 2), psem[sp]).wait(),
                        lambda: None,
                    )
                compute(si, sg, sp)
                lax.cond(t + NI < per_worker, lambda: start_idx(t + NI, si), lambda: None)
                lax.cond(
                    t + NG < per_worker,
                    lambda: start_gather(t + NG, (b + NG) % NI, sg),
                    lambda: None,
                )
                pltpu.async_copy(pbuf[sp], out_slice(t), psem[sp])
            return 0

        lax.fori_loop(0, n_outer, body, 0)
        pltpu.make_async_copy(pbuf[0], out_slice(per_worker - 2), psem[0]).wait()
        pltpu.make_async_copy(pbuf[1], out_slice(per_worker - 1), psem[1]).wait()

    return k2


def kernel(vars_, index):
    b, s = index.shape
    idx = index.T.reshape(-1).astype(jnp.int32)
    tbl_t = vars_.T
    tail_t = lax.slice(tbl_t, (0, VOCAB_N - 128), (EMBED_D, VOCAB_N))
    w = _make_transpose()(tbl_t, tail_t)
    p = _make_gather(idx.shape[0])(w, idx)
    return p.transpose(2, 0, 1)


# speedup vs baseline: 1.8334x; 1.8334x over previous
import functools

import jax
import jax.numpy as jnp
from jax import lax
from jax.experimental import pallas as pl
from jax.experimental.pallas import tpu as pltpu
from jax.experimental.pallas import tpu_sc as plsc

NORM = 1.0
MEAN = 0.0
VOCAB_N = 1000000
EMBED_D = 64
NWORK = 32
PAIR_ROWS = VOCAB_N // 2
FULL_VBLKS = VOCAB_N // 128
MAIN_VBLKS = (FULL_VBLKS // NWORK) * NWORK
CHUNK = 128


def _iota16():
    return jnp.arange(16, dtype=jnp.int32)


def _make_transpose():
    mesh = plsc.VectorSubcoreMesh(core_axis_name="c", subcore_axis_name="s")

    NB = 4

    @functools.partial(
        pl.kernel,
        out_type=jax.ShapeDtypeStruct((PAIR_ROWS, 128), jnp.float32),
        mesh=mesh,
        scratch_types=[
            [pltpu.VMEM((64, 128), jnp.float32) for _ in range(NB)],
            [pltpu.VMEM((64, 128), jnp.float32) for _ in range(2)],
            [pltpu.SemaphoreType.DMA for _ in range(NB)],
            [pltpu.SemaphoreType.DMA for _ in range(2)],
        ],
        compiler_params=pltpu.CompilerParams(use_tc_tiling_on_sc=True, needs_layout_passes=False),
    )
    def k1(tbl_t, tail_t, w_out, bb, wb, bsem, wsem):
        wid = lax.axis_index("s") * 2 + lax.axis_index("c")
        rowv = tuple(_iota16() + (16 * m) for m in range(4))
        n_main = MAIN_VBLKS // NWORK

        def in_slice(t):
            off = pl.multiple_of((t * NWORK + wid) * 128, 128)
            return tbl_t.at[:, pl.ds(off, 128)]

        def out_slice(t):
            off = pl.multiple_of((t * NWORK + wid) * 64, 64)
            return w_out.at[pl.ds(off, 64), :]

        def start_in(t, s):
            pltpu.async_copy(in_slice(t), bb[s], bsem[s])
            return None

        iota = _iota16()

        def transpose_block(b, wb_):
            @plsc.parallel_loop(0, 16, unroll=2)
            def _(r):
                rot = jnp.bitwise_and(iota + r, 15)
                scol = 2 * iota
                for p0 in range(0, 64, 16):
                    for half in range(2):
                        for m in range(4):
                            v = plsc.load_gather(
                                b, [rot + 16 * m, scol + (2 * p0 + half)]
                            )
                            plsc.store_scatter(
                                wb_, [iota + p0, rot + (64 * half + 16 * m)], v
                            )

        for t0 in range(NB):
            start_in(t0, t0)

        def body(tt, _):
            base = tt * NB
            for b in range(NB):
                t = base + b
                sw = b % 2
                pltpu.make_async_copy(in_slice(t), bb[b], bsem[b]).wait()
                if b >= 2:
                    pltpu.make_async_copy(wb[sw], out_slice(t - 2), wsem[sw]).wait()
                else:
                    lax.cond(
                        tt > 0,
                        lambda: pltpu.make_async_copy(wb[sw], out_slice(t - 2), wsem[sw]).wait(),
                        lambda: None,
                    )
                transpose_block(bb[b], wb[sw])
                lax.cond(t + NB < n_main, lambda: start_in(t + NB, b), lambda: None)
                pltpu.async_copy(wb[sw], out_slice(t), wsem[sw])
            return 0

        lax.fori_loop(0, n_main // NB, body, 0)
        pltpu.make_async_copy(wb[0], out_slice(n_main - 2), wsem[0]).wait()
        pltpu.make_async_copy(wb[1], out_slice(n_main - 1), wsem[1]).wait()

        @pl.when(wid < FULL_VBLKS - MAIN_VBLKS)
        def _():
            vblk = MAIN_VBLKS + wid
            pltpu.sync_copy(tbl_t.at[:, pl.ds(vblk * 128, 128)], bb[0])
            transpose_block(bb[0], wb[0])
            pltpu.sync_copy(wb[0], w_out.at[pl.ds(vblk * 64, 64), :])

        @pl.when(wid == 4)
        def _():
            pltpu.sync_copy(tail_t, bb[1])
            transpose_block(bb[1], wb[1])
            pltpu.sync_copy(wb[1].at[pl.ds(32, 32), :], w_out.at[pl.ds(FULL_VBLKS * 64, 32), :])

    return k1


def _make_gather(total: int):
    n_chunks = total // CHUNK
    per_worker = n_chunks // NWORK
    mesh = plsc.VectorSubcoreMesh(core_axis_name="c", subcore_axis_name="s")

    NG = 4
    NI = 8

    @functools.partial(
        pl.kernel,
        out_type=jax.ShapeDtypeStruct((26, EMBED_D, 16384), jnp.float32),
        mesh=mesh,
        scratch_types=[
            [pltpu.VMEM((CHUNK,), jnp.int32) for _ in range(NI)],
            [pltpu.VMEM((CHUNK,), jnp.int32) for _ in range(NG)],
            [pltpu.VMEM((CHUNK, 128), jnp.float32) for _ in range(NG)],
            [pltpu.VMEM((EMBED_D, 128), jnp.float32) for _ in range(2)],
            [pltpu.SemaphoreType.DMA for _ in range(NI)],
            [pltpu.SemaphoreType.DMA for _ in range(NG)],
            [pltpu.SemaphoreType.DMA for _ in range(2)],
        ],
        compiler_params=pltpu.CompilerParams(use_tc_tiling_on_sc=True, needs_layout_passes=False),
    )
    def k2(w_in, idx_in, p_out, ix, pr, g, pbuf, isem, gsem, psem):
        wid = lax.axis_index("s") * 2 + lax.axis_index("c")
        rowv = tuple(_iota16() + (16 * m) for m in range(4))

        def chunk_id(t):
            return t * NWORK + wid

        def idx_slice(t):
            off = pl.multiple_of(chunk_id(t) * CHUNK, CHUNK)
            return idx_in.at[pl.ds(off, CHUNK)]

        def out_slice(t):
            c = chunk_id(t)
            j = c // 128
            off = pl.multiple_of(lax.rem(c, 128) * 128, 128)
            return p_out.at[j, :, pl.ds(off, 128)]

        def start_idx(t, si):
            pltpu.async_copy(idx_slice(t), ix[si], isem[si])
            return None

        def start_gather(t, si, sg):
            pltpu.make_async_copy(idx_slice(t), ix[si], isem[si]).wait()
            for m in range(8):
                x = ix[si][pl.ds(16 * m, 16)]
                pr[sg][pl.ds(16 * m, 16)] = jnp.right_shift(x, 1)
            pltpu.async_copy(w_in.at[pr[sg]], g[sg], gsem[sg])
            return None

        def compute(si, sg, sp):
            colbase = []
            for m in range(8):
                x = ix[si][pl.ds(16 * m, 16)]
                colbase.append(jnp.left_shift(jnp.bitwise_and(x, 1), 6))

            @plsc.parallel_loop(0, EMBED_D, unroll=8)
            def _(d):
                for m in range(8):
                    rv = rowv[m % 4] + (64 if m >= 4 else 0)
                    v = plsc.load_gather(g[sg], [rv, colbase[m] + d])
                    pbuf[sp][d, pl.ds(16 * m, 16)] = v

        for t0 in range(NI):
            start_idx(t0, t0)
        for t0 in range(NG):
            start_gather(t0, t0, t0)

        n_outer = per_worker // NI

        def body(tt, _):
            base = tt * NI
            for b in range(NI):
                t = base + b
                si, sg, sp = b, b % NG, b % 2
                pltpu.make_async_copy(w_in.at[pr[sg]], g[sg], gsem[sg]).wait()
                if b >= 2:
                    pltpu.make_async_copy(pbuf[sp], out_slice(t - 2), psem[sp]).wait()
                else:
                    lax.cond(
                        tt > 0,
                        lambda: pltpu.make_async_copy(pbuf[sp], out_slice(t - 2), psem[sp]).wait(),
                        lambda: None,
                    )
                compute(si, sg, sp)
                lax.cond(t + NI < per_worker, lambda: start_idx(t + NI, si), lambda: None)
                lax.cond(
                    t + NG < per_worker,
                    lambda: start_gather(t + NG, (b + NG) % NI, sg),
                    lambda: None,
                )
                pltpu.async_copy(pbuf[sp], out_slice(t), psem[sp])
            return 0

        lax.fori_loop(0, n_outer, body, 0)
        pltpu.make_async_copy(pbuf[0], out_slice(per_worker - 2), psem[0]).wait()
        pltpu.make_async_copy(pbuf[1], out_slice(per_worker - 1), psem[1]).wait()

    return k2


def kernel(vars_, index):
    b, s = index.shape
    idx = index.T.reshape(-1).astype(jnp.int32)
    tbl_t = vars_.T
    tail_t = lax.slice(tbl_t, (0, VOCAB_N - 128), (EMBED_D, VOCAB_N))
    w = _make_transpose()(tbl_t, tail_t)
    p = _make_gather(idx.shape[0])(w, idx)
    return p.transpose(2, 0, 1)


# speedup vs baseline: 2.4763x; 1.3507x over previous
import functools

import jax
import jax.numpy as jnp
from jax import lax
from jax.experimental import pallas as pl
from jax.experimental.pallas import tpu as pltpu
from jax.experimental.pallas import tpu_sc as plsc

NORM = 1.0
MEAN = 0.0
VOCAB_N = 1000000
EMBED_D = 64
NWORK = 32
PAIR_ROWS = VOCAB_N // 2
FULL_VBLKS = VOCAB_N // 128
MAIN_VBLKS = (FULL_VBLKS // NWORK) * NWORK
CHUNK = 128


def _iota16():
    return jnp.arange(16, dtype=jnp.int32)


def _make_transpose():
    mesh = plsc.VectorSubcoreMesh(core_axis_name="c", subcore_axis_name="s")

    NB = 4

    @functools.partial(
        pl.kernel,
        out_type=jax.ShapeDtypeStruct((PAIR_ROWS, 128), jnp.float32),
        mesh=mesh,
        scratch_types=[
            [pltpu.VMEM((64, 128), jnp.float32) for _ in range(NB)],
            [pltpu.VMEM((64, 128), jnp.float32) for _ in range(2)],
            [pltpu.SemaphoreType.DMA for _ in range(NB)],
            [pltpu.SemaphoreType.DMA for _ in range(2)],
        ],
        compiler_params=pltpu.CompilerParams(use_tc_tiling_on_sc=True, needs_layout_passes=False),
    )
    def k1(tbl_t, tail_t, w_out, bb, wb, bsem, wsem):
        wid = lax.axis_index("s") * 2 + lax.axis_index("c")
        rowv = tuple(_iota16() + (16 * m) for m in range(4))
        n_main = MAIN_VBLKS // NWORK

        def in_slice(t):
            off = pl.multiple_of((t * NWORK + wid) * 128, 128)
            return tbl_t.at[:, pl.ds(off, 128)]

        def out_slice(t):
            off = pl.multiple_of((t * NWORK + wid) * 64, 64)
            return w_out.at[pl.ds(off, 64), :]

        def start_in(t, s):
            pltpu.async_copy(in_slice(t), bb[s], bsem[s])
            return None

        iota = _iota16()

        def transpose_block(b, wb_):
            @plsc.parallel_loop(0, 16, unroll=2)
            def _(r):
                rot = jnp.bitwise_and(iota + r, 15)
                scol = 2 * iota
                for p0 in range(0, 64, 16):
                    for half in range(2):
                        for m in range(4):
                            v = plsc.load_gather(
                                b, [rot + 16 * m, scol + (2 * p0 + half)]
                            )
                            plsc.store_scatter(
                                wb_, [iota + p0, rot + (64 * half + 16 * m)], v
                            )

        for t0 in range(NB):
            start_in(t0, t0)

        def body(tt, _):
            base = tt * NB
            for b in range(NB):
                t = base + b
                sw = b % 2
                pltpu.make_async_copy(in_slice(t), bb[b], bsem[b]).wait()
                if b >= 2:
                    pltpu.make_async_copy(wb[sw], out_slice(t - 2), wsem[sw]).wait()
                else:
                    lax.cond(
                        tt > 0,
                        lambda: pltpu.make_async_copy(wb[sw], out_slice(t - 2), wsem[sw]).wait(),
                        lambda: None,
                    )
                transpose_block(bb[b], wb[sw])
                lax.cond(t + NB < n_main, lambda: start_in(t + NB, b), lambda: None)
                pltpu.async_copy(wb[sw], out_slice(t), wsem[sw])
            return 0

        lax.fori_loop(0, n_main // NB, body, 0)
        pltpu.make_async_copy(wb[0], out_slice(n_main - 2), wsem[0]).wait()
        pltpu.make_async_copy(wb[1], out_slice(n_main - 1), wsem[1]).wait()

        @pl.when(wid < FULL_VBLKS - MAIN_VBLKS)
        def _():
            vblk = MAIN_VBLKS + wid
            pltpu.sync_copy(tbl_t.at[:, pl.ds(vblk * 128, 128)], bb[0])
            transpose_block(bb[0], wb[0])
            pltpu.sync_copy(wb[0], w_out.at[pl.ds(vblk * 64, 64), :])

        @pl.when(wid == 4)
        def _():
            pltpu.sync_copy(tail_t, bb[1])
            transpose_block(bb[1], wb[1])
            pltpu.sync_copy(wb[1].at[pl.ds(32, 32), :], w_out.at[pl.ds(FULL_VBLKS * 64, 32), :])

    return k1


def _make_gather(total: int):
    n_chunks = total // CHUNK
    per_worker = n_chunks // NWORK
    mesh = plsc.VectorSubcoreMesh(core_axis_name="c", subcore_axis_name="s")

    NG = 4
    NI = 8

    @functools.partial(
        pl.kernel,
        out_type=jax.ShapeDtypeStruct((26, EMBED_D, 16384), jnp.float32),
        mesh=mesh,
        scratch_types=[
            [pltpu.VMEM((CHUNK,), jnp.int32) for _ in range(NI)],
            [pltpu.VMEM((CHUNK,), jnp.int32) for _ in range(NG)],
            [pltpu.VMEM((CHUNK, 128), jnp.float32) for _ in range(NG)],
            [pltpu.VMEM((EMBED_D, 128), jnp.float32) for _ in range(2)],
            [pltpu.SemaphoreType.DMA for _ in range(NI)],
            [pltpu.SemaphoreType.DMA for _ in range(NG)],
            [pltpu.SemaphoreType.DMA for _ in range(2)],
        ],
        compiler_params=pltpu.CompilerParams(use_tc_tiling_on_sc=True, needs_layout_passes=False),
    )
    def k2(w_in, idx_in, p_out, ix, pr, g, pbuf, isem, gsem, psem):
        wid = lax.axis_index("s") * 2 + lax.axis_index("c")
        rowv = tuple(_iota16() + (16 * m) for m in range(4))

        def chunk_id(t):
            return t * NWORK + wid

        def idx_slice(t):
            off = pl.multiple_of(chunk_id(t) * CHUNK, CHUNK)
            return idx_in.at[pl.ds(off, CHUNK)]

        def out_slice(t):
            c = chunk_id(t)
            j = c // 128
            off = pl.multiple_of(lax.rem(c, 128) * 128, 128)
            return p_out.at[j, :, pl.ds(off, 128)]

        def start_idx(t, si):
            pltpu.async_copy(idx_slice(t), ix[si], isem[si])
            return None

        def start_gather(t, si, sg):
            pltpu.make_async_copy(idx_slice(t), ix[si], isem[si]).wait()
            for m in range(8):
                x = ix[si][pl.ds(16 * m, 16)]
                pr[sg][pl.ds(16 * m, 16)] = jnp.right_shift(x, 1)
            pltpu.async_copy(w_in.at[pr[sg]], g[sg], gsem[sg])
            return None

        iota = _iota16()

        def compute(si, sg, sp):
            colbase = []
            for m in range(8):
                x = ix[si][pl.ds(16 * m, 16)]
                colbase.append(jnp.left_shift(jnp.bitwise_and(x, 1), 6))

            @plsc.parallel_loop(0, 16, unroll=2)
            def _(r):
                rot = jnp.bitwise_and(iota + r, 15)
                rots = [rot + d0 for d0 in range(0, EMBED_D, 16)]
                for m in range(8):
                    grow = iota + 16 * m
                    for di in range(4):
                        v = plsc.load_gather(g[sg], [grow, colbase[m] + rots[di]])
                        plsc.store_scatter(pbuf[sp], [rots[di], grow], v)

        for t0 in range(NI):
            start_idx(t0, t0)
        for t0 in range(NG):
            start_gather(t0, t0, t0)

        n_outer = per_worker // NI

        def body(tt, _):
            base = tt * NI
            for b in range(NI):
                t = base + b
                si, sg, sp = b, b % NG, b % 2
                pltpu.make_async_copy(w_in.at[pr[sg]], g[sg], gsem[sg]).wait()
                if b >= 2:
                    pltpu.make_async_copy(pbuf[sp], out_slice(t - 2), psem[sp]).wait()
                else:
                    lax.cond(
                        tt > 0,
                        lambda: pltpu.make_async_copy(pbuf[sp], out_slice(t - 2), psem[sp]).wait(),
                        lambda: None,
                    )
                compute(si, sg, sp)
                lax.cond(t + NI < per_worker, lambda: start_idx(t + NI, si), lambda: None)
                lax.cond(
                    t + NG < per_worker,
                    lambda: start_gather(t + NG, (b + NG) % NI, sg),
                    lambda: None,
                )
                pltpu.async_copy(pbuf[sp], out_slice(t), psem[sp])
            return 0

        lax.fori_loop(0, n_outer, body, 0)
        pltpu.make_async_copy(pbuf[0], out_slice(per_worker - 2), psem[0]).wait()
        pltpu.make_async_copy(pbuf[1], out_slice(per_worker - 1), psem[1]).wait()

    return k2


def kernel(vars_, index):
    b, s = index.shape
    idx = index.T.reshape(-1).astype(jnp.int32)
    tbl_t = vars_.T
    tail_t = lax.slice(tbl_t, (0, VOCAB_N - 128), (EMBED_D, VOCAB_N))
    w = _make_transpose()(tbl_t, tail_t)
    p = _make_gather(idx.shape[0])(w, idx)
    return p.transpose(2, 0, 1)
